# Initial kernel scaffold; baseline (speedup 1.0000x reference)
import jax
import jax.numpy as jnp
from jax.experimental import pallas as pl


def kernel(z, E):
    raise NotImplementedError("write your pallas kernel here")


# speedup vs baseline: 1.9434x; 1.9434x over previous
import functools

import jax
import jax.numpy as jnp
from jax.experimental import pallas as pl
from jax.experimental.pallas import tpu as pltpu

_NE = 512
_D = 32
_BETA = 0.25
_ROWS = 65536
_R = 1024
_NT = _ROWS // _R


def _vq_tile(x_ref, e_ref, d_ref, inds_ref, zq_ref, loss_ref):
    i = pl.program_id(0)
    x = x_ref[...]
    e = e_ref[...]

    x2 = jnp.sum(x * x, axis=1, keepdims=True)
    e2_full = jax.lax.dot_general(
        jnp.ones((8, _D), jnp.float32), e * e,
        (((1,), (1,)), ((), ())),
        precision=jax.lax.Precision.HIGHEST,
        preferred_element_type=jnp.float32)
    e2 = e2_full[0:1, :]
    ze = jax.lax.dot_general(
        x, e, (((1,), (1,)), ((), ())),
        preferred_element_type=jnp.float32)
    d = (x2 + e2) - 2.0 * ze
    d_ref[...] = d

    dmin = jnp.min(d, axis=1, keepdims=True)
    lane = jax.lax.broadcasted_iota(jnp.int32, (_R, _NE), 1)
    idx = jnp.min(jnp.where(d == dmin, lane, _NE), axis=1,
                  keepdims=True)
    inds_ref[...] = idx

    oh = (lane == idx).astype(jnp.float32)
    zq = jax.lax.dot_general(
        oh, e, (((1,), (0,)), ((), ())),
        precision=jax.lax.Precision.HIGHEST,
        preferred_element_type=jnp.float32)
    zq_ref[...] = x + (zq - x)

    diff = zq - x
    part = jnp.sum(diff * diff).reshape(1, 1)

    @pl.when(i == 0)
    def _():
        loss_ref[...] = part

    @pl.when(jnp.logical_and(i > 0, i < _NT - 1))
    def _():
        loss_ref[...] = loss_ref[...] + part

    @pl.when(i == _NT - 1)
    def _():
        total = loss_ref[...] + part
        m = total / jnp.float32(_ROWS * _D)
        loss_ref[...] = m + _BETA * m


@functools.partial(jax.jit, static_argnames=("interpret",))
def kernel(z, E, interpret=False):
    B, C, T, H, W = z.shape
    flat = jnp.transpose(z, (0, 2, 3, 4, 1)).reshape(-1, C)

    d, inds, zq_flat, loss = pl.pallas_call(
        _vq_tile,
        grid=(_NT,),
        in_specs=[
            pl.BlockSpec((_R, _D), lambda i: (i, 0)),
            pl.BlockSpec((_NE, _D), lambda i: (0, 0)),
        ],
        out_specs=[
            pl.BlockSpec((_R, _NE), lambda i: (i, 0)),
            pl.BlockSpec((_R, 1), lambda i: (i, 0)),
            pl.BlockSpec((_R, _D), lambda i: (i, 0)),
            pl.BlockSpec((1, 1), lambda i: (0, 0)),
        ],
        out_shape=[
            jax.ShapeDtypeStruct((_ROWS, _NE), jnp.float32),
            jax.ShapeDtypeStruct((_ROWS, 1), jnp.int32),
            jax.ShapeDtypeStruct((_ROWS, _D), jnp.float32),
            jax.ShapeDtypeStruct((1, 1), jnp.float32),
        ],
        interpret=interpret,
    )(flat, E)

    z_q_st = jnp.transpose(zq_flat.reshape(B, T, H, W, C), (0, 4, 1, 2, 3))
    inds_out = inds.reshape(B, T, H, W)
    return z_q_st, loss.reshape(()), inds_out, d
